# baseline (speedup 1.0000x reference)
import jax
import jax.numpy as jnp
from jax import lax
from jax.experimental import pallas as pl
from jax.experimental.pallas import tpu as pltpu
from jax.experimental.pallas import tpu_sc as plsc

N_Y = 100000
N_X = 32768
E = 1600000
C = 32
NUM_FREQ = 32
MAX_POS = 10000.0
D_EMB = 3 * NUM_FREQ * 2

NC, NS = 2, 16
NW = NC * NS
GK = 128
GG = 96
SUP = 2 * GG
CH2 = 131
EPW = 2 * SUP * CH2
E_PAD = NW * EPW
CH = EPW // GK
ACC_R = 34816

_mesh = plsc.VectorSubcoreMesh(
    core_axis_name="c", subcore_axis_name="s", num_cores=NC, num_subcores=NS)
_sc_params = pltpu.CompilerParams(use_tc_tiling_on_sc=False,
                                  needs_layout_passes=False)


def _pack_bf16_pair(a, b):
    wa = lax.bitcast_convert_type(a.astype(jnp.bfloat16).astype(jnp.float32),
                                  jnp.uint32)
    wb = lax.bitcast_convert_type(b.astype(jnp.bfloat16).astype(jnp.float32),
                                  jnp.uint32)
    return lax.bitcast_convert_type(wa | (wb >> 16), jnp.float32)


def _unpack_bf16_pair(w):
    u = lax.bitcast_convert_type(w, jnp.uint32)
    a = lax.bitcast_convert_type(u & jnp.uint32(0xFFFF0000), jnp.float32)
    b = lax.bitcast_convert_type(u << 16, jnp.float32)
    return a, b


def _node_y_body(c_ref, fy_ref, s_ref, ph_ref, w_ref, b_ref, o_ref):
    ang = jnp.dot(c_ref[...], s_ref[...], preferred_element_type=jnp.float32)
    emb = jnp.sin(ang + ph_ref[...])
    h = (jnp.dot(emb, w_ref[...], preferred_element_type=jnp.float32)
         + b_ref[...])
    o_ref[:, 0:64] = _pack_bf16_pair(h[:, :64], h[:, 64:])
    o_ref[:, 64:96] = fy_ref[...]


def _node_x_body(c_ref, s_ref, ph_ref, w_ref, o_ref):
    ang = jnp.dot(c_ref[...], s_ref[...], preferred_element_type=jnp.float32)
    emb = jnp.sin(ang + ph_ref[...])
    o_ref[...] = jnp.dot(emb, w_ref[...], preferred_element_type=jnp.float32)


def _gather_body(yhf_hbm, xh_hbm, nidx_hbm, qidx_hbm, ga_hbm,
                 nva, qva, yba, xba, nvb, qvb, ybb, xbb,
                 semga, semgb, semwa, semwb):
    c = lax.axis_index("c")
    s = lax.axis_index("s")
    base = (s * NC + c) * EPW

    def fire_gathers(nv, qv, yb, xb, sem):
        cps = []
        for k in range(SUP // GG):
            sl = pl.ds(k * GG, GG)
            cps.append(pltpu.async_copy(yhf_hbm.at[nv.at[sl]], yb.at[sl], sem))
            cps.append(pltpu.async_copy(xh_hbm.at[qv.at[sl]], xb.at[sl], sem))
        return cps

    def drain_wb(yb, sem):
        pltpu.make_async_copy(yb, ga_hbm.at[pl.ds(0, SUP)], sem).wait()

    def fuse_add(yb, xb):
        def edge(r, carry):
            for w in range(4):
                yw = lax.bitcast_convert_type(yb[r, pl.ds(w * 16, 16)],
                                              jnp.uint32)
                hi = lax.bitcast_convert_type(yw & jnp.uint32(0xFFFF0000),
                                              jnp.float32)
                lo = lax.bitcast_convert_type(yw << 16, jnp.float32)
                s1 = hi + xb[r, pl.ds(w * 16, 16)]
                s2 = lo + xb[r, pl.ds(64 + w * 16, 16)]
                packed = plsc.pack(s2, s1, format=plsc.PackFormat.INTERLEAVED)
                yb[r, pl.ds(w * 16, 16)] = plsc.bitcast(packed, jnp.float32)
            return carry
        lax.fori_loop(0, SUP, edge, 0)

    def body(j, carry):
        sta = pl.multiple_of(base + (2 * j) * SUP, 8)
        stb = pl.multiple_of(base + (2 * j + 1) * SUP, 8)
        pltpu.sync_copy(nidx_hbm.at[pl.ds(sta, SUP)], nva)
        pltpu.sync_copy(qidx_hbm.at[pl.ds(sta, SUP)], qva)
        pltpu.sync_copy(nidx_hbm.at[pl.ds(stb, SUP)], nvb)
        pltpu.sync_copy(qidx_hbm.at[pl.ds(stb, SUP)], qvb)

        @pl.when(j > 0)
        def _():
            drain_wb(yba, semwa)
        ga = fire_gathers(nva, qva, yba, xba, semga)

        @pl.when(j > 0)
        def _():
            drain_wb(ybb, semwb)
        gb = fire_gathers(nvb, qvb, ybb, xbb, semgb)

        for cp in ga:
            cp.wait()
        fuse_add(yba, xba)
        pltpu.async_copy(yba, ga_hbm.at[pl.ds(sta, SUP)], semwa)
        for cp in gb:
            cp.wait()
        fuse_add(ybb, xbb)
        pltpu.async_copy(ybb, ga_hbm.at[pl.ds(stb, SUP)], semwb)
        return carry

    lax.fori_loop(0, CH2, body, 0)
    drain_wb(yba, semwa)
    drain_wb(ybb, semwb)


def _gelu_bf16(x):
    xb = x.astype(jnp.bfloat16)
    c1 = jnp.bfloat16(0.7978845608028654)
    c2 = jnp.bfloat16(0.7978845608028654 * 0.044715)
    half = jnp.bfloat16(0.5)
    one = jnp.bfloat16(1.0)
    u = xb * xb
    t = jnp.tanh(xb * (c1 + c2 * u))
    return (half * xb) * (one + t)


def _mlp_body(gyf_ref, w2_ref, b2_ref, w3_ref, b3_ref,
              w4_ref, b4_ref, o_ref):
    ya, yb = _unpack_bf16_pair(gyf_ref[:, 0:64])
    h1a = _gelu_bf16(ya)
    h1b = _gelu_bf16(yb)
    t2 = (jnp.dot(h1a, w2_ref[0:64, :].astype(jnp.bfloat16),
                  preferred_element_type=jnp.float32)
          + jnp.dot(h1b, w2_ref[64:128, :].astype(jnp.bfloat16),
                    preferred_element_type=jnp.float32)
          + b2_ref[...])
    h2 = _gelu_bf16(t2)
    h3 = _gelu_bf16(
        jnp.dot(h2, w3_ref[...].astype(jnp.bfloat16),
                preferred_element_type=jnp.float32)
        + b3_ref[...])
    k = (jnp.dot(h3, w4_ref[...].astype(jnp.bfloat16),
                 preferred_element_type=jnp.float32)
         + b4_ref[...])
    o_ref[...] = k * gyf_ref[:, 64:96]


def _scatter_body(msgl_hbm, qidx_hbm, out_hbm, qv, mb, zb, t128, acc, sem):
    c = lax.axis_index("c")
    s = lax.axis_index("s")
    base = (s * NC + c) * EPW
    zero16 = jnp.zeros((16,), jnp.float32)

    def zb_row(r, carry):
        zb[r, 0:16] = zero16
        zb[r, 16:32] = zero16
        return carry
    lax.fori_loop(0, 128, zb_row, 0)
    zrows = ACC_R // NS
    def zcp(i, carry):
        pltpu.sync_copy(zb, acc.at[pl.ds(s * zrows + i * 128, 128)])
        return carry
    lax.fori_loop(0, zrows // 128, zcp, 0)
    plsc.subcore_barrier()

    def body(i, carry):
        st = pl.multiple_of(base + i * GK, 8)
        pltpu.sync_copy(qidx_hbm.at[pl.ds(st, GK)], qv)
        pltpu.sync_copy(msgl_hbm.at[pl.ds(st, GK)], mb)
        pltpu.sync_copy(mb, acc.at[qv], add=True)
        return carry

    lax.fori_loop(0, CH, body, 0)
    plsc.subcore_barrier()

    orows = N_X // NS
    def dump(i, carry):
        r0 = s * orows + i * 128
        pltpu.sync_copy(acc.at[pl.ds(r0, 128)], mb)
        def expand(r, carry2):
            t128[r, 0:16] = mb[r, 0:16]
            t128[r, 16:32] = mb[r, 16:32]
            return carry2
        lax.fori_loop(0, 128, expand, 0)
        pltpu.sync_copy(t128, out_hbm.at[c].at[pl.ds(r0, 128)])
        return carry
    lax.fori_loop(0, orows // 128, dump, 0)


def _combine_body(p_ref, o_ref):
    o_ref[...] = p_ref[0, :, :C] + p_ref[1, :, :C]


def kernel(y, x, f_y, neighbors_index, query_index,
           W1, b1, W2, b2, W3, b3, W4, b4):
    f32 = jnp.float32

    k = jnp.arange(NUM_FREQ, dtype=f32)
    freqs = 1.0 / (MAX_POS ** (k / NUM_FREQ))
    eye3 = jnp.eye(3, dtype=f32)
    ff = jnp.concatenate([freqs, freqs])
    S = (eye3[:, :, None] * ff[None, None, :]).reshape(3, D_EMB)
    S = jnp.concatenate([S, jnp.zeros((5, D_EMB), f32)], axis=0)
    ph_row = jnp.concatenate([jnp.zeros((NUM_FREQ,), f32),
                              jnp.full((NUM_FREQ,), jnp.pi / 2, f32)])
    ph = jnp.tile(ph_row, 3)[None, :]

    y8 = jnp.concatenate([y, jnp.zeros((N_Y, 5), f32)], axis=1)
    x8 = jnp.concatenate([x, jnp.zeros((N_X, 5), f32)], axis=1)

    RBY = 400
    YHF = pl.pallas_call(
        _node_y_body,
        grid=(N_Y // RBY,),
        in_specs=[
            pl.BlockSpec((RBY, 8), lambda i: (i, 0)),
            pl.BlockSpec((RBY, C), lambda i: (i, 0)),
            pl.BlockSpec((8, D_EMB), lambda i: (0, 0)),
            pl.BlockSpec((1, D_EMB), lambda i: (0, 0)),
            pl.BlockSpec((D_EMB, 128), lambda i: (0, 0)),
            pl.BlockSpec((1, 128), lambda i: (0, 0)),
        ],
        out_specs=pl.BlockSpec((RBY, 96), lambda i: (i, 0)),
        out_shape=jax.ShapeDtypeStruct((N_Y, 96), f32),
    )(y8, f_y, S, ph, W1[D_EMB:], b1[None, :])

    RBX = 512
    XH = pl.pallas_call(
        _node_x_body,
        grid=(N_X // RBX,),
        in_specs=[
            pl.BlockSpec((RBX, 8), lambda i: (i, 0)),
            pl.BlockSpec((8, D_EMB), lambda i: (0, 0)),
            pl.BlockSpec((1, D_EMB), lambda i: (0, 0)),
            pl.BlockSpec((D_EMB, 128), lambda i: (0, 0)),
        ],
        out_specs=pl.BlockSpec((RBX, 128), lambda i: (i, 0)),
        out_shape=jax.ShapeDtypeStruct((N_X, 128), f32),
    )(x8, S, ph, W1[:D_EMB])

    npad = jnp.concatenate([neighbors_index.astype(jnp.int32),
                            jnp.zeros((E_PAD - E,), jnp.int32)])
    qpad = jnp.concatenate([query_index.astype(jnp.int32),
                            jnp.full((E_PAD - E,), N_X, jnp.int32)])
    XHp = jnp.concatenate([XH, jnp.zeros((8, 128), f32)], axis=0)

    gather = pl.kernel(
        _gather_body,
        out_type=jax.ShapeDtypeStruct((E_PAD, 96), f32),
        mesh=_mesh,
        compiler_params=_sc_params,
        scratch_types=[
            pltpu.VMEM((SUP,), jnp.int32),
            pltpu.VMEM((SUP,), jnp.int32),
            pltpu.VMEM((SUP, 96), f32),
            pltpu.VMEM((SUP, 128), f32),
            pltpu.VMEM((SUP,), jnp.int32),
            pltpu.VMEM((SUP,), jnp.int32),
            pltpu.VMEM((SUP, 96), f32),
            pltpu.VMEM((SUP, 128), f32),
            pltpu.SemaphoreType.DMA,
            pltpu.SemaphoreType.DMA,
            pltpu.SemaphoreType.DMA,
            pltpu.SemaphoreType.DMA,
        ],
    )
    GA = gather(YHF, XHp, npad, qpad)

    BE = 2048
    MSG = pl.pallas_call(
        _mlp_body,
        grid=(E_PAD // BE,),
        in_specs=[
            pl.BlockSpec((BE, 96), lambda i: (i, 0)),
            pl.BlockSpec((128, 256), lambda i: (0, 0)),
            pl.BlockSpec((1, 256), lambda i: (0, 0)),
            pl.BlockSpec((256, 128), lambda i: (0, 0)),
            pl.BlockSpec((1, 128), lambda i: (0, 0)),
            pl.BlockSpec((128, C), lambda i: (0, 0)),
            pl.BlockSpec((1, C), lambda i: (0, 0)),
        ],
        out_specs=pl.BlockSpec((BE, C), lambda i: (i, 0)),
        out_shape=jax.ShapeDtypeStruct((E_PAD, C), f32),
    )(GA, W2, b2[None, :], W3, b3[None, :], W4, b4[None, :])

    scatter = pl.kernel(
        _scatter_body,
        out_type=jax.ShapeDtypeStruct((NC, N_X, 128), f32),
        mesh=_mesh,
        compiler_params=_sc_params,
        scratch_types=[
            pltpu.VMEM((GK,), jnp.int32),
            pltpu.VMEM((GK, C), f32),
            pltpu.VMEM((128, C), f32),
            pltpu.VMEM((128, 128), f32),
            pltpu.VMEM_SHARED((ACC_R, C), f32),
            pltpu.SemaphoreType.DMA,
        ],
    )
    parts = scatter(MSG, qpad)

    RBO = 512
    out = pl.pallas_call(
        _combine_body,
        grid=(N_X // RBO,),
        in_specs=[pl.BlockSpec((NC, RBO, 128), lambda i: (0, i, 0))],
        out_specs=pl.BlockSpec((RBO, C), lambda i: (i, 0)),
        out_shape=jax.ShapeDtypeStruct((N_X, C), f32),
    )(parts)
    return out


# speedup vs baseline: 1.1674x; 1.1674x over previous
import jax
import jax.numpy as jnp
from jax import lax
from jax.experimental import pallas as pl
from jax.experimental.pallas import tpu as pltpu
from jax.experimental.pallas import tpu_sc as plsc

N_Y = 100000
N_X = 32768
E = 1600000
C = 32
NUM_FREQ = 32
MAX_POS = 10000.0
D_EMB = 3 * NUM_FREQ * 2

NC, NS = 2, 16
NW = NC * NS
GK = 128
GG = 96
SUP = 2 * GG
CH2 = 131
EPW = 2 * SUP * CH2
E_PAD = NW * EPW
CH = EPW // GK
ACC_R = 34816

_mesh = plsc.VectorSubcoreMesh(
    core_axis_name="c", subcore_axis_name="s", num_cores=NC, num_subcores=NS)
_sc_params = pltpu.CompilerParams(use_tc_tiling_on_sc=False,
                                  needs_layout_passes=False)


def _pack_bf16_pair(a, b):
    wa = lax.bitcast_convert_type(a.astype(jnp.bfloat16).astype(jnp.float32),
                                  jnp.uint32)
    wb = lax.bitcast_convert_type(b.astype(jnp.bfloat16).astype(jnp.float32),
                                  jnp.uint32)
    return lax.bitcast_convert_type(wa | (wb >> 16), jnp.float32)


def _unpack_bf16_pair(w):
    u = lax.bitcast_convert_type(w, jnp.uint32)
    a = lax.bitcast_convert_type(u & jnp.uint32(0xFFFF0000), jnp.float32)
    b = lax.bitcast_convert_type(u << 16, jnp.float32)
    return a, b


def _node_y_body(c_ref, fy_ref, s_ref, ph_ref, w_ref, b_ref, o_ref):
    ang = jnp.dot(c_ref[...], s_ref[...], preferred_element_type=jnp.float32)
    emb = jnp.sin(ang + ph_ref[...])
    h = (jnp.dot(emb, w_ref[...], preferred_element_type=jnp.float32)
         + b_ref[...])
    o_ref[:, 0:64] = _pack_bf16_pair(h[:, :64], h[:, 64:])
    o_ref[:, 64:96] = fy_ref[...]
    o_ref[:, 96:128] = jnp.zeros_like(fy_ref[...])


def _node_x_body(c_ref, s_ref, ph_ref, w_ref, o_ref):
    ang = jnp.dot(c_ref[...], s_ref[...], preferred_element_type=jnp.float32)
    emb = jnp.sin(ang + ph_ref[...])
    o_ref[...] = jnp.dot(emb, w_ref[...], preferred_element_type=jnp.float32)


def _gather_body(yhf_hbm, xh_hbm, nidx_hbm, qidx_hbm, ga_hbm,
                 nva, qva, yba, xba, nvb, qvb, ybb, xbb,
                 semga, semgb, semwa, semwb):
    c = lax.axis_index("c")
    s = lax.axis_index("s")
    base = (s * NC + c) * EPW

    def fire_gathers(nv, qv, yb, xb, sem):
        cps = []
        for k in range(SUP // GG):
            sl = pl.ds(k * GG, GG)
            cps.append(pltpu.async_copy(yhf_hbm.at[nv.at[sl]], yb.at[sl], sem))
            cps.append(pltpu.async_copy(xh_hbm.at[qv.at[sl]], xb.at[sl], sem))
        return cps

    def drain_wb(yb, sem):
        pltpu.make_async_copy(yb, ga_hbm.at[pl.ds(0, SUP)], sem).wait()

    def fuse_add(yb, xb):
        def edge(r, carry):
            for w in range(4):
                yw = lax.bitcast_convert_type(yb[r, pl.ds(w * 16, 16)],
                                              jnp.uint32)
                hi = lax.bitcast_convert_type(yw & jnp.uint32(0xFFFF0000),
                                              jnp.float32)
                lo = lax.bitcast_convert_type(yw << 16, jnp.float32)
                s1 = hi + xb[r, pl.ds(w * 16, 16)]
                s2 = lo + xb[r, pl.ds(64 + w * 16, 16)]
                packed = plsc.pack(s2, s1, format=plsc.PackFormat.INTERLEAVED)
                yb[r, pl.ds(w * 16, 16)] = plsc.bitcast(packed, jnp.float32)
            return carry
        lax.fori_loop(0, SUP, edge, 0)

    def body(j, carry):
        sta = pl.multiple_of(base + (2 * j) * SUP, 8)
        stb = pl.multiple_of(base + (2 * j + 1) * SUP, 8)
        pltpu.sync_copy(nidx_hbm.at[pl.ds(sta, SUP)], nva)
        pltpu.sync_copy(qidx_hbm.at[pl.ds(sta, SUP)], qva)
        pltpu.sync_copy(nidx_hbm.at[pl.ds(stb, SUP)], nvb)
        pltpu.sync_copy(qidx_hbm.at[pl.ds(stb, SUP)], qvb)

        @pl.when(j > 0)
        def _():
            drain_wb(yba, semwa)
        ga = fire_gathers(nva, qva, yba, xba, semga)

        @pl.when(j > 0)
        def _():
            drain_wb(ybb, semwb)
        gb = fire_gathers(nvb, qvb, ybb, xbb, semgb)

        for cp in ga:
            cp.wait()
        fuse_add(yba, xba)
        pltpu.async_copy(yba, ga_hbm.at[pl.ds(sta, SUP)], semwa)
        for cp in gb:
            cp.wait()
        fuse_add(ybb, xbb)
        pltpu.async_copy(ybb, ga_hbm.at[pl.ds(stb, SUP)], semwb)
        return carry

    lax.fori_loop(0, CH2, body, 0)
    drain_wb(yba, semwa)
    drain_wb(ybb, semwb)


def _gelu_bf16(x):
    xb = x.astype(jnp.bfloat16)
    c1 = jnp.bfloat16(0.7978845608028654)
    c2 = jnp.bfloat16(0.7978845608028654 * 0.044715)
    half = jnp.bfloat16(0.5)
    one = jnp.bfloat16(1.0)
    u = xb * xb
    t = jnp.tanh(xb * (c1 + c2 * u))
    return (half * xb) * (one + t)


def _mlp_body(gyf_ref, w2_ref, b2_ref, w3_ref, b3_ref,
              w4_ref, b4_ref, o_ref):
    ya, yb = _unpack_bf16_pair(gyf_ref[:, 0:64])
    h1a = _gelu_bf16(ya)
    h1b = _gelu_bf16(yb)
    t2 = (jnp.dot(h1a, w2_ref[0:64, :].astype(jnp.bfloat16),
                  preferred_element_type=jnp.float32)
          + jnp.dot(h1b, w2_ref[64:128, :].astype(jnp.bfloat16),
                    preferred_element_type=jnp.float32)
          + b2_ref[...])
    h2 = _gelu_bf16(t2)
    h3 = _gelu_bf16(
        jnp.dot(h2, w3_ref[...].astype(jnp.bfloat16),
                preferred_element_type=jnp.float32)
        + b3_ref[...])
    k = (jnp.dot(h3, w4_ref[...].astype(jnp.bfloat16),
                 preferred_element_type=jnp.float32)
         + b4_ref[...])
    o_ref[...] = k * gyf_ref[:, 64:96]


def _scatter_body(msgl_hbm, qidx_hbm, out_hbm, qv, mb, zb, t128, acc, sem):
    c = lax.axis_index("c")
    s = lax.axis_index("s")
    base = (s * NC + c) * EPW
    zero16 = jnp.zeros((16,), jnp.float32)

    def zb_row(r, carry):
        zb[r, 0:16] = zero16
        zb[r, 16:32] = zero16
        return carry
    lax.fori_loop(0, 128, zb_row, 0)
    zrows = ACC_R // NS
    def zcp(i, carry):
        pltpu.sync_copy(zb, acc.at[pl.ds(s * zrows + i * 128, 128)])
        return carry
    lax.fori_loop(0, zrows // 128, zcp, 0)
    plsc.subcore_barrier()

    def body(i, carry):
        st = pl.multiple_of(base + i * GK, 8)
        pltpu.sync_copy(qidx_hbm.at[pl.ds(st, GK)], qv)
        pltpu.sync_copy(msgl_hbm.at[pl.ds(st, GK)], mb)
        pltpu.sync_copy(mb, acc.at[qv], add=True)
        return carry

    lax.fori_loop(0, CH, body, 0)
    plsc.subcore_barrier()

    orows = N_X // NS
    def dump(i, carry):
        r0 = s * orows + i * 128
        pltpu.sync_copy(acc.at[pl.ds(r0, 128)], mb)
        def expand(r, carry2):
            t128[r, 0:16] = mb[r, 0:16]
            t128[r, 16:32] = mb[r, 16:32]
            return carry2
        lax.fori_loop(0, 128, expand, 0)
        pltpu.sync_copy(t128, out_hbm.at[c].at[pl.ds(r0, 128)])
        return carry
    lax.fori_loop(0, orows // 128, dump, 0)


def _combine_body(p_ref, o_ref):
    o_ref[...] = p_ref[0, :, :C] + p_ref[1, :, :C]


def kernel(y, x, f_y, neighbors_index, query_index,
           W1, b1, W2, b2, W3, b3, W4, b4):
    f32 = jnp.float32

    k = jnp.arange(NUM_FREQ, dtype=f32)
    freqs = 1.0 / (MAX_POS ** (k / NUM_FREQ))
    eye3 = jnp.eye(3, dtype=f32)
    ff = jnp.concatenate([freqs, freqs])
    S = (eye3[:, :, None] * ff[None, None, :]).reshape(3, D_EMB)
    S = jnp.concatenate([S, jnp.zeros((5, D_EMB), f32)], axis=0)
    ph_row = jnp.concatenate([jnp.zeros((NUM_FREQ,), f32),
                              jnp.full((NUM_FREQ,), jnp.pi / 2, f32)])
    ph = jnp.tile(ph_row, 3)[None, :]

    y8 = jnp.concatenate([y, jnp.zeros((N_Y, 5), f32)], axis=1)
    x8 = jnp.concatenate([x, jnp.zeros((N_X, 5), f32)], axis=1)

    RBY = 400
    YHF = pl.pallas_call(
        _node_y_body,
        grid=(N_Y // RBY,),
        in_specs=[
            pl.BlockSpec((RBY, 8), lambda i: (i, 0)),
            pl.BlockSpec((RBY, C), lambda i: (i, 0)),
            pl.BlockSpec((8, D_EMB), lambda i: (0, 0)),
            pl.BlockSpec((1, D_EMB), lambda i: (0, 0)),
            pl.BlockSpec((D_EMB, 128), lambda i: (0, 0)),
            pl.BlockSpec((1, 128), lambda i: (0, 0)),
        ],
        out_specs=pl.BlockSpec((RBY, 128), lambda i: (i, 0)),
        out_shape=jax.ShapeDtypeStruct((N_Y, 128), f32),
    )(y8, f_y, S, ph, W1[D_EMB:], b1[None, :])

    RBX = 512
    XH = pl.pallas_call(
        _node_x_body,
        grid=(N_X // RBX,),
        in_specs=[
            pl.BlockSpec((RBX, 8), lambda i: (i, 0)),
            pl.BlockSpec((8, D_EMB), lambda i: (0, 0)),
            pl.BlockSpec((1, D_EMB), lambda i: (0, 0)),
            pl.BlockSpec((D_EMB, 128), lambda i: (0, 0)),
        ],
        out_specs=pl.BlockSpec((RBX, 128), lambda i: (i, 0)),
        out_shape=jax.ShapeDtypeStruct((N_X, 128), f32),
    )(x8, S, ph, W1[:D_EMB])

    npad = jnp.concatenate([neighbors_index.astype(jnp.int32),
                            jnp.zeros((E_PAD - E,), jnp.int32)])
    qpad = jnp.concatenate([query_index.astype(jnp.int32),
                            jnp.full((E_PAD - E,), N_X, jnp.int32)])
    XHp = jnp.concatenate([XH, jnp.zeros((8, 128), f32)], axis=0)

    gather = pl.kernel(
        _gather_body,
        out_type=jax.ShapeDtypeStruct((E_PAD, 128), f32),
        mesh=_mesh,
        compiler_params=_sc_params,
        scratch_types=[
            pltpu.VMEM((SUP,), jnp.int32),
            pltpu.VMEM((SUP,), jnp.int32),
            pltpu.VMEM((SUP, 128), f32),
            pltpu.VMEM((SUP, 128), f32),
            pltpu.VMEM((SUP,), jnp.int32),
            pltpu.VMEM((SUP,), jnp.int32),
            pltpu.VMEM((SUP, 128), f32),
            pltpu.VMEM((SUP, 128), f32),
            pltpu.SemaphoreType.DMA,
            pltpu.SemaphoreType.DMA,
            pltpu.SemaphoreType.DMA,
            pltpu.SemaphoreType.DMA,
        ],
    )
    GA = gather(YHF, XHp, npad, qpad)

    BE = 2048
    MSG = pl.pallas_call(
        _mlp_body,
        grid=(E_PAD // BE,),
        in_specs=[
            pl.BlockSpec((BE, 128), lambda i: (i, 0)),
            pl.BlockSpec((128, 256), lambda i: (0, 0)),
            pl.BlockSpec((1, 256), lambda i: (0, 0)),
            pl.BlockSpec((256, 128), lambda i: (0, 0)),
            pl.BlockSpec((1, 128), lambda i: (0, 0)),
            pl.BlockSpec((128, C), lambda i: (0, 0)),
            pl.BlockSpec((1, C), lambda i: (0, 0)),
        ],
        out_specs=pl.BlockSpec((BE, C), lambda i: (i, 0)),
        out_shape=jax.ShapeDtypeStruct((E_PAD, C), f32),
    )(GA, W2, b2[None, :], W3, b3[None, :], W4, b4[None, :])

    scatter = pl.kernel(
        _scatter_body,
        out_type=jax.ShapeDtypeStruct((NC, N_X, 128), f32),
        mesh=_mesh,
        compiler_params=_sc_params,
        scratch_types=[
            pltpu.VMEM((GK,), jnp.int32),
            pltpu.VMEM((GK, C), f32),
            pltpu.VMEM((128, C), f32),
            pltpu.VMEM((128, 128), f32),
            pltpu.VMEM_SHARED((ACC_R, C), f32),
            pltpu.SemaphoreType.DMA,
        ],
    )
    parts = scatter(MSG, qpad)

    RBO = 512
    out = pl.pallas_call(
        _combine_body,
        grid=(N_X // RBO,),
        in_specs=[pl.BlockSpec((NC, RBO, 128), lambda i: (0, i, 0))],
        out_specs=pl.BlockSpec((RBO, C), lambda i: (i, 0)),
        out_shape=jax.ShapeDtypeStruct((N_X, C), f32),
    )(parts)
    return out
